# Initial kernel scaffold; baseline (speedup 1.0000x reference)
import jax
import jax.numpy as jnp
from jax.experimental import pallas as pl


def kernel(x, edge_index, edge_attr, Wq, bq, Wk, bk, Wv, bv, We, Wskip, bskip):
    raise NotImplementedError("write your pallas kernel here")


# speedup vs baseline: 13.2047x; 13.2047x over previous
import functools
import math

import jax
import jax.numpy as jnp
from jax import lax
from jax.experimental import pallas as pl
from jax.experimental.pallas import tpu as pltpu
from jax.experimental.pallas import tpu_sc as plsc

_N = 10000
_E = 320000
_DIN = 128
_HC = 128
_NC = 2
_NS = 16
_CH = 16
_NW = _NC * _NS
_EP_PAD = ((_E + _CH * _NW - 1) // (_CH * _NW)) * (_CH * _NW)
_NP = ((_N + _CH - 1) // _CH) * _CH
_TRASH = _N + 8
_NPAD_X = 10240


def _pack2(lo_f32, hi_f32):
    lo = lax.bitcast_convert_type(lo_f32.astype(jnp.bfloat16),
                                  jnp.uint16).astype(jnp.int32)
    hi = lax.bitcast_convert_type(hi_f32.astype(jnp.bfloat16),
                                  jnp.uint16).astype(jnp.int32)
    return lax.bitwise_or(lo, lax.shift_left(hi, 16))


def _prep_body(x_ref, w_ref, b_ref, q_ref, kv_ref, sk_ref):
    acc = jnp.dot(x_ref[...], w_ref[...],
                  preferred_element_type=jnp.float32) + b_ref[...]
    q_ref[...] = acc[:, 0:128]
    k = acc[:, 128:256]
    v = acc[:, 256:384]
    kv_ref[...] = jnp.concatenate(
        [_pack2(k[:, 0:64], k[:, 64:128]),
         _pack2(v[:, 0:64], v[:, 64:128])], axis=1)
    sk_ref[...] = acc[:, 384:512]


def _ep_body(a_ref, w_ref, o_ref):
    e = jnp.dot(a_ref[...], w_ref[...], preferred_element_type=jnp.float32)
    o_ref[...] = _pack2(e[:, 0:64], e[:, 64:128])


def _edge_kernel_body(q_hbm, kv_hbm, ep_hbm, src_hbm, dst_hbm,
                      out_hbm, sout_hbm,
                      srcva, dstva, qva, kvva, eva,
                      srcvb, dstvb, qvb, kvvb, evb,
                      rowv, slocal, idx160, acc, s_acc,
                      sqa, ska, sea, sqb, skb, seb):
    c = lax.axis_index("c")
    s = lax.axis_index("s")
    wid = c * _NS + s
    iot = lax.iota(jnp.int32, 16)

    def _zrow(i, _):
        for k in range(8):
            rowv[i, 16 * k:16 * k + 16] = jnp.zeros((16,), jnp.float32)
        return 0

    lax.fori_loop(0, _CH, _zrow, 0)

    def _zs(i, _):
        for k in range(8):
            slocal[i, 16 * k:16 * k + 16] = jnp.zeros((16,), jnp.float32)
        return 0

    lax.fori_loop(0, 160, _zs, 0)

    def _zi(t, _):
        idx160[pl.ds(t * 16, 16)] = iot + t * 16
        return 0

    lax.fori_loop(0, 10, _zi, 0)

    def _zacc(t, _):
        j = s + t * _NS

        @pl.when(j < _NP // _CH)
        def _():
            rb = pl.multiple_of(j * _CH, 8)
            pltpu.sync_copy(rowv, acc.at[pl.ds(rb, _CH)])

        return 0

    lax.fori_loop(0, _NP // _CH // _NS + 1, _zacc, 0)

    @pl.when(s < 10)
    def _():
        rb = pl.multiple_of(s * 16, 8)
        pltpu.sync_copy(rowv.at[pl.ds(0, 16)], s_acc.at[pl.ds(rb, 16)])

    plsc.subcore_barrier()

    inv_sqrt_c = jnp.float32(1.0 / math.sqrt(64.0))
    bufs_a = (srcva, dstva, qva, kvva, eva, sqa, ska, sea)
    bufs_b = (srcvb, dstvb, qvb, kvvb, evb, sqb, skb, seb)

    def prefetch(tch, bufs):
        srcx, dstx, qx, kvx, ex, sq, sk_, se = bufs
        base = pl.multiple_of((wid + tch * _NW) * _CH, 8)
        pltpu.sync_copy(src_hbm.at[pl.ds(base, _CH)], srcx)
        pltpu.sync_copy(dst_hbm.at[pl.ds(base, _CH)], dstx)
        pltpu.async_copy(q_hbm.at[dstx], qx, sq)
        pltpu.async_copy(kv_hbm.at[srcx], kvx, sk_)
        pltpu.async_copy(ep_hbm.at[pl.ds(base, _CH)], ex, se)

    def compute(bufs):
        srcx, dstx, qx, kvx, ex, sq, sk_, se = bufs
        pltpu.make_async_copy(q_hbm.at[dstx], qx, sq).wait()
        pltpu.make_async_copy(kv_hbm.at[srcx], kvx, sk_).wait()
        pltpu.make_async_copy(ep_hbm.at[pl.ds(0, _CH)], ex, se).wait()
        for g in range(_CH // 16):
            r = g * 16 + iot

            z16 = jnp.zeros((16,), jnp.float32)

            def _dot_body(i, acc):
                accl = list(acc)
                c0 = i * 4
                for u in range(4):
                    cv = jnp.broadcast_to(c0 + u, (16,)).astype(jnp.int32)
                    q0 = plsc.load_gather(qx, [r, cv])
                    q1 = plsc.load_gather(qx, [r, cv + 64])
                    kp = plsc.load_gather(kvx, [r, cv])
                    epk = plsc.load_gather(ex, [r, cv])
                    k0, k1 = plsc.unpack(plsc.bitcast(kp, jnp.bfloat16),
                                         format=plsc.PackFormat.INTERLEAVED)
                    e0, e1 = plsc.unpack(plsc.bitcast(epk, jnp.bfloat16),
                                         format=plsc.PackFormat.INTERLEAVED)
                    accl[u] = accl[u] + q0 * (k0 + e0)
                    accl[4 + u] = accl[4 + u] + q1 * (k1 + e1)
                return tuple(accl)

            dot_acc = plsc.parallel_loop(
                0, 16, unroll=4, carry=(z16,) * 8)(_dot_body)
            a0 = (dot_acc[0] + dot_acc[1]) + (dot_acc[2] + dot_acc[3])
            a1 = (dot_acc[4] + dot_acc[5]) + (dot_acc[6] + dot_acc[7])
            w0 = jnp.exp(a0 * inv_sqrt_c)
            w1 = jnp.exp(a1 * inv_sqrt_c)

            d16 = dstx[pl.ds(g * 16, 16)]
            f0 = d16 * 2
            row0 = lax.shift_right_logical(f0, 7)
            col0 = lax.bitwise_and(f0, 127)
            plsc.addupdate_scatter(slocal, [row0, col0], w0)
            plsc.addupdate_scatter(slocal, [row0, col0 + 1], w1)

            def _msg_body(c0):
                cv = jnp.broadcast_to(c0, (16,)).astype(jnp.int32)
                vp = plsc.load_gather(kvx, [r, cv + 64])
                epk = plsc.load_gather(ex, [r, cv])
                v0, v1 = plsc.unpack(plsc.bitcast(vp, jnp.bfloat16),
                                     format=plsc.PackFormat.INTERLEAVED)
                e0, e1 = plsc.unpack(plsc.bitcast(epk, jnp.bfloat16),
                                     format=plsc.PackFormat.INTERLEAVED)
                plsc.store_scatter(rowv, [r, cv], (v0 + e0) * w0)
                plsc.store_scatter(rowv, [r, cv + 64], (v1 + e1) * w1)

            plsc.parallel_loop(0, 64, unroll=8)(_msg_body)
        pltpu.sync_copy(rowv, acc.at[dstx], add=True)

    n_per_w = _EP_PAD // _CH // _NW
    prefetch(0, bufs_a)

    def step(t, _):
        prefetch(2 * t + 1, bufs_b)
        compute(bufs_a)
        prefetch(2 * t + 2, bufs_a)
        compute(bufs_b)
        return 0

    lax.fori_loop(0, (n_per_w - 1) // 2, step, 0)
    compute(bufs_a)

    pltpu.sync_copy(slocal, s_acc.at[idx160], add=True)
    plsc.subcore_barrier()

    @pl.when(s < 10)
    def _():
        rb = pl.multiple_of(s * 16, 8)
        pltpu.sync_copy(s_acc.at[pl.ds(rb, 16)],
                        sout_hbm.at[c, pl.ds(rb, 16)])

    def _dump(t, _):
        j = s + t * _NS

        @pl.when(j < _NP // _CH)
        def _():
            rb = pl.multiple_of(j * _CH, 8)
            pltpu.sync_copy(acc.at[pl.ds(rb, _CH)],
                            out_hbm.at[c, pl.ds(rb, _CH)])

        return 0

    lax.fori_loop(0, _NP // _CH // _NS + 1, _dump, 0)


_edge_kernel = functools.partial(
    pl.kernel,
    compiler_params=pltpu.CompilerParams(needs_layout_passes=False),
    out_type=(jax.ShapeDtypeStruct((_NC, _NP, 128), jnp.float32),
              jax.ShapeDtypeStruct((_NC, 160, 128), jnp.float32)),
    mesh=plsc.VectorSubcoreMesh(core_axis_name="c", subcore_axis_name="s"),
    scratch_types=[
        pltpu.VMEM((_CH,), jnp.int32),
        pltpu.VMEM((_CH,), jnp.int32),
        pltpu.VMEM((_CH, 128), jnp.float32),
        pltpu.VMEM((_CH, 128), jnp.int32),
        pltpu.VMEM((_CH, 64), jnp.int32),
        pltpu.VMEM((_CH,), jnp.int32),
        pltpu.VMEM((_CH,), jnp.int32),
        pltpu.VMEM((_CH, 128), jnp.float32),
        pltpu.VMEM((_CH, 128), jnp.int32),
        pltpu.VMEM((_CH, 64), jnp.int32),
        pltpu.VMEM((_CH, 128), jnp.float32),
        pltpu.VMEM((160, 128), jnp.float32),
        pltpu.VMEM((160,), jnp.int32),
        pltpu.VMEM_SHARED((_NP, 128), jnp.float32),
        pltpu.VMEM_SHARED((160, 128), jnp.float32),
        pltpu.SemaphoreType.DMA,
        pltpu.SemaphoreType.DMA,
        pltpu.SemaphoreType.DMA,
        pltpu.SemaphoreType.DMA,
        pltpu.SemaphoreType.DMA,
        pltpu.SemaphoreType.DMA,
    ],
)(_edge_kernel_body)


def _fin_body(p_ref, s_ref, sk_ref, o_ref):
    a = p_ref[0] + p_ref[1]
    sv = s_ref[0] + s_ref[1]
    s0 = sv[:, 0:1]
    s1 = sv[:, 1:2]
    o_ref[...] = jnp.concatenate(
        [a[:, 0:64] / (s0 + 1e-16), a[:, 64:128] / (s1 + 1e-16)],
        axis=1) + sk_ref[...]


def kernel(x, edge_index, edge_attr, Wq, bq, Wk, bk, Wv, bv, We, Wskip, bskip):
    w_all = jnp.concatenate([Wq, Wk, Wv, Wskip], axis=1)
    b_all = jnp.concatenate([bq, bk, bv, bskip]).reshape(1, 512)
    x_pad = jnp.pad(x, ((0, _NPAD_X - _N), (0, 0)))

    q, kv, sk = pl.pallas_call(
        _prep_body,
        grid=(5,),
        in_specs=[
            pl.BlockSpec((2048, 128), lambda i: (i, 0)),
            pl.BlockSpec((128, 512), lambda i: (0, 0)),
            pl.BlockSpec((1, 512), lambda i: (0, 0)),
        ],
        out_specs=[
            pl.BlockSpec((2048, 128), lambda i: (i, 0)),
            pl.BlockSpec((2048, 128), lambda i: (i, 0)),
            pl.BlockSpec((2048, 128), lambda i: (i, 0)),
        ],
        out_shape=[
            jax.ShapeDtypeStruct((_NPAD_X, 128), jnp.float32),
            jax.ShapeDtypeStruct((_NPAD_X, 128), jnp.int32),
            jax.ShapeDtypeStruct((_NPAD_X, 128), jnp.float32),
        ],
    )(x_pad, w_all, b_all)

    ep = pl.pallas_call(
        _ep_body,
        grid=(40,),
        in_specs=[
            pl.BlockSpec((8000, 16), lambda i: (i, 0)),
            pl.BlockSpec((16, 128), lambda i: (0, 0)),
        ],
        out_specs=pl.BlockSpec((8000, 64), lambda i: (i, 0)),
        out_shape=jax.ShapeDtypeStruct((_E, 64), jnp.int32),
    )(edge_attr, We)

    ep_pad = jnp.pad(ep, ((0, _EP_PAD - _E), (0, 0)))
    src_pad = jnp.pad(edge_index[0], (0, _EP_PAD - _E))
    dst_pad = jnp.pad(edge_index[1], (0, _EP_PAD - _E),
                      constant_values=_TRASH)

    partial, s_out = _edge_kernel(q, kv, ep_pad, src_pad, dst_pad)
    partial = partial[:, :_N]
    s_out = s_out.reshape(_NC, 160 * 128)[:, :2 * _N].reshape(_NC, _N, 2)

    out = pl.pallas_call(
        _fin_body,
        grid=(10,),
        in_specs=[
            pl.BlockSpec((2, 1000, 128), lambda i: (0, i, 0)),
            pl.BlockSpec((2, 1000, 2), lambda i: (0, i, 0)),
            pl.BlockSpec((1000, 128), lambda i: (i, 0)),
        ],
        out_specs=pl.BlockSpec((1000, 128), lambda i: (i, 0)),
        out_shape=jax.ShapeDtypeStruct((_N, _HC), jnp.float32),
    )(partial, s_out, sk[:_N])
    return out


# speedup vs baseline: 15.8374x; 1.1994x over previous
import functools
import math

import jax
import jax.numpy as jnp
from jax import lax
from jax.experimental import pallas as pl
from jax.experimental.pallas import tpu as pltpu
from jax.experimental.pallas import tpu_sc as plsc

_N = 10000
_E = 320000
_DIN = 128
_HC = 128
_NC = 2
_NS = 16
_CH = 16
_NW = _NC * _NS
_EP_PAD = ((_E + _CH * _NW - 1) // (_CH * _NW)) * (_CH * _NW)
_EPW = _EP_PAD // _NW
_IB = 32
_IBE = _IB * _CH
_IDX_PAD = _EP_PAD + _IBE
_NP = ((_N + _CH - 1) // _CH) * _CH
_TRASH = _N + 8
_NPAD_X = 10240


def _pack2(lo_f32, hi_f32):
    lo = lax.bitcast_convert_type(lo_f32.astype(jnp.bfloat16),
                                  jnp.uint16).astype(jnp.int32)
    hi = lax.bitcast_convert_type(hi_f32.astype(jnp.bfloat16),
                                  jnp.uint16).astype(jnp.int32)
    return lax.bitwise_or(lo, lax.shift_left(hi, 16))


def _prep_body(x_ref, w_ref, b_ref, q_ref, kv_ref, sk_ref):
    acc = jnp.dot(x_ref[...], w_ref[...],
                  preferred_element_type=jnp.float32) + b_ref[...]
    q_ref[...] = acc[:, 0:128]
    k = acc[:, 128:256]
    v = acc[:, 256:384]
    kv_ref[...] = jnp.concatenate(
        [_pack2(k[:, 0:64], k[:, 64:128]),
         _pack2(v[:, 0:64], v[:, 64:128])], axis=1)
    sk_ref[...] = acc[:, 384:512]


def _ep_body(a_ref, w_ref, o_ref):
    e = jnp.dot(a_ref[...], w_ref[...], preferred_element_type=jnp.float32)
    o_ref[...] = _pack2(e[:, 0:64], e[:, 64:128])


def _edge_kernel_body(q_hbm, kv_hbm, ep_hbm, src_hbm, dst_hbm,
                      out_hbm, sout_hbm,
                      srcva, dstva, qva, kvva, eva,
                      srcvb, dstvb, qvb, kvvb, evb,
                      srcblk, dstblk,
                      rowv, slocal, idx160, acc, s_acc,
                      sqa, ska, sea, sqb, skb, seb):
    c = lax.axis_index("c")
    s = lax.axis_index("s")
    wid = c * _NS + s
    iot = lax.iota(jnp.int32, 16)

    def _zrow(i, _):
        for k in range(8):
            rowv[i, 16 * k:16 * k + 16] = jnp.zeros((16,), jnp.float32)
        return 0

    lax.fori_loop(0, _CH, _zrow, 0)

    def _zs(i, _):
        for k in range(8):
            slocal[i, 16 * k:16 * k + 16] = jnp.zeros((16,), jnp.float32)
        return 0

    lax.fori_loop(0, 160, _zs, 0)

    def _zi(t, _):
        idx160[pl.ds(t * 16, 16)] = iot + t * 16
        return 0

    lax.fori_loop(0, 10, _zi, 0)

    def _zacc(t, _):
        j = s + t * _NS

        @pl.when(j < _NP // _CH)
        def _():
            rb = pl.multiple_of(j * _CH, 8)
            pltpu.sync_copy(rowv, acc.at[pl.ds(rb, _CH)])

        return 0

    lax.fori_loop(0, _NP // _CH // _NS + 1, _zacc, 0)

    @pl.when(s < 10)
    def _():
        rb = pl.multiple_of(s * 16, 8)
        pltpu.sync_copy(rowv.at[pl.ds(0, 16)], s_acc.at[pl.ds(rb, 16)])

    plsc.subcore_barrier()

    inv_sqrt_c = jnp.float32(1.0 / math.sqrt(64.0))
    bufs_a = (srcva, dstva, qva, kvva, eva, sqa, ska, sea)
    bufs_b = (srcvb, dstvb, qvb, kvvb, evb, sqb, skb, seb)

    def prefetch(tch, bufs):
        srcx, dstx, qx, kvx, ex, sq, sk_, se = bufs
        t_loc = lax.bitwise_and(tch, _IB - 1)

        @pl.when(t_loc == 0)
        def _():
            bb = pl.multiple_of(wid * _EPW + (tch // _IB) * _IBE, 8)
            pltpu.sync_copy(src_hbm.at[pl.ds(bb, _IBE)], srcblk)
            pltpu.sync_copy(dst_hbm.at[pl.ds(bb, _IBE)], dstblk)

        off = pl.multiple_of(t_loc * _CH, 8)
        srcx[pl.ds(0, 16)] = srcblk[pl.ds(off, 16)]
        dstx[pl.ds(0, 16)] = dstblk[pl.ds(off, 16)]
        base = pl.multiple_of(wid * _EPW + tch * _CH, 8)
        pltpu.async_copy(q_hbm.at[dstx], qx, sq)
        pltpu.async_copy(kv_hbm.at[srcx], kvx, sk_)
        pltpu.async_copy(ep_hbm.at[pl.ds(base, _CH)], ex, se)

    def compute(bufs):
        srcx, dstx, qx, kvx, ex, sq, sk_, se = bufs
        pltpu.make_async_copy(q_hbm.at[dstx], qx, sq).wait()
        pltpu.make_async_copy(kv_hbm.at[srcx], kvx, sk_).wait()
        pltpu.make_async_copy(ep_hbm.at[pl.ds(0, _CH)], ex, se).wait()
        for g in range(_CH // 16):
            r = g * 16 + iot

            z16 = jnp.zeros((16,), jnp.float32)

            def _dot_body(i, acc):
                accl = list(acc)
                c0 = i * 4
                for u in range(4):
                    cv = jnp.broadcast_to(c0 + u, (16,)).astype(jnp.int32)
                    q0 = plsc.load_gather(qx, [r, cv])
                    q1 = plsc.load_gather(qx, [r, cv + 64])
                    kp = plsc.load_gather(kvx, [r, cv])
                    epk = plsc.load_gather(ex, [r, cv])
                    k0, k1 = plsc.unpack(plsc.bitcast(kp, jnp.bfloat16),
                                         format=plsc.PackFormat.INTERLEAVED)
                    e0, e1 = plsc.unpack(plsc.bitcast(epk, jnp.bfloat16),
                                         format=plsc.PackFormat.INTERLEAVED)
                    accl[u] = accl[u] + q0 * (k0 + e0)
                    accl[4 + u] = accl[4 + u] + q1 * (k1 + e1)
                return tuple(accl)

            dot_acc = plsc.parallel_loop(
                0, 16, unroll=4, carry=(z16,) * 8)(_dot_body)
            a0 = (dot_acc[0] + dot_acc[1]) + (dot_acc[2] + dot_acc[3])
            a1 = (dot_acc[4] + dot_acc[5]) + (dot_acc[6] + dot_acc[7])
            w0 = jnp.exp(a0 * inv_sqrt_c)
            w1 = jnp.exp(a1 * inv_sqrt_c)

            d16 = dstx[pl.ds(g * 16, 16)]
            f0 = d16 * 2
            row0 = lax.shift_right_logical(f0, 7)
            col0 = lax.bitwise_and(f0, 127)
            plsc.addupdate_scatter(slocal, [row0, col0], w0)
            plsc.addupdate_scatter(slocal, [row0, col0 + 1], w1)

            def _msg_body(c0):
                cv = jnp.broadcast_to(c0, (16,)).astype(jnp.int32)
                vp = plsc.load_gather(kvx, [r, cv + 64])
                epk = plsc.load_gather(ex, [r, cv])
                v0, v1 = plsc.unpack(plsc.bitcast(vp, jnp.bfloat16),
                                     format=plsc.PackFormat.INTERLEAVED)
                e0, e1 = plsc.unpack(plsc.bitcast(epk, jnp.bfloat16),
                                     format=plsc.PackFormat.INTERLEAVED)
                plsc.store_scatter(rowv, [r, cv], (v0 + e0) * w0)
                plsc.store_scatter(rowv, [r, cv + 64], (v1 + e1) * w1)

            plsc.parallel_loop(0, 64, unroll=8)(_msg_body)
        pltpu.sync_copy(rowv, acc.at[dstx], add=True)

    n_per_w = _EP_PAD // _CH // _NW
    prefetch(0, bufs_a)

    def step(t, _):
        prefetch(2 * t + 1, bufs_b)
        compute(bufs_a)
        prefetch(2 * t + 2, bufs_a)
        compute(bufs_b)
        return 0

    lax.fori_loop(0, (n_per_w - 1) // 2, step, 0)
    compute(bufs_a)

    pltpu.sync_copy(slocal, s_acc.at[idx160], add=True)
    plsc.subcore_barrier()

    @pl.when(s < 10)
    def _():
        rb = pl.multiple_of(s * 16, 8)
        pltpu.sync_copy(s_acc.at[pl.ds(rb, 16)],
                        sout_hbm.at[c, pl.ds(rb, 16)])

    def _dump(t, _):
        j = s + t * _NS

        @pl.when(j < _NP // _CH)
        def _():
            rb = pl.multiple_of(j * _CH, 8)
            pltpu.sync_copy(acc.at[pl.ds(rb, _CH)],
                            out_hbm.at[c, pl.ds(rb, _CH)])

        return 0

    lax.fori_loop(0, _NP // _CH // _NS + 1, _dump, 0)


_edge_kernel = functools.partial(
    pl.kernel,
    compiler_params=pltpu.CompilerParams(needs_layout_passes=False),
    out_type=(jax.ShapeDtypeStruct((_NC, _NP, 128), jnp.float32),
              jax.ShapeDtypeStruct((_NC, 160, 128), jnp.float32)),
    mesh=plsc.VectorSubcoreMesh(core_axis_name="c", subcore_axis_name="s"),
    scratch_types=[
        pltpu.VMEM((_CH,), jnp.int32),
        pltpu.VMEM((_CH,), jnp.int32),
        pltpu.VMEM((_CH, 128), jnp.float32),
        pltpu.VMEM((_CH, 128), jnp.int32),
        pltpu.VMEM((_CH, 64), jnp.int32),
        pltpu.VMEM((_CH,), jnp.int32),
        pltpu.VMEM((_CH,), jnp.int32),
        pltpu.VMEM((_CH, 128), jnp.float32),
        pltpu.VMEM((_CH, 128), jnp.int32),
        pltpu.VMEM((_CH, 64), jnp.int32),
        pltpu.VMEM((_IBE,), jnp.int32),
        pltpu.VMEM((_IBE,), jnp.int32),
        pltpu.VMEM((_CH, 128), jnp.float32),
        pltpu.VMEM((160, 128), jnp.float32),
        pltpu.VMEM((160,), jnp.int32),
        pltpu.VMEM_SHARED((_NP, 128), jnp.float32),
        pltpu.VMEM_SHARED((160, 128), jnp.float32),
        pltpu.SemaphoreType.DMA,
        pltpu.SemaphoreType.DMA,
        pltpu.SemaphoreType.DMA,
        pltpu.SemaphoreType.DMA,
        pltpu.SemaphoreType.DMA,
        pltpu.SemaphoreType.DMA,
    ],
)(_edge_kernel_body)


def _fin_body(p_ref, s_ref, sk_ref, o_ref):
    a = p_ref[0] + p_ref[1]
    sv = s_ref[0] + s_ref[1]
    s0 = sv[:, 0:1]
    s1 = sv[:, 1:2]
    o_ref[...] = jnp.concatenate(
        [a[:, 0:64] / (s0 + 1e-16), a[:, 64:128] / (s1 + 1e-16)],
        axis=1) + sk_ref[...]


def kernel(x, edge_index, edge_attr, Wq, bq, Wk, bk, Wv, bv, We, Wskip, bskip):
    w_all = jnp.concatenate([Wq, Wk, Wv, Wskip], axis=1)
    b_all = jnp.concatenate([bq, bk, bv, bskip]).reshape(1, 512)
    x_pad = jnp.pad(x, ((0, _NPAD_X - _N), (0, 0)))

    q, kv, sk = pl.pallas_call(
        _prep_body,
        grid=(5,),
        in_specs=[
            pl.BlockSpec((2048, 128), lambda i: (i, 0)),
            pl.BlockSpec((128, 512), lambda i: (0, 0)),
            pl.BlockSpec((1, 512), lambda i: (0, 0)),
        ],
        out_specs=[
            pl.BlockSpec((2048, 128), lambda i: (i, 0)),
            pl.BlockSpec((2048, 128), lambda i: (i, 0)),
            pl.BlockSpec((2048, 128), lambda i: (i, 0)),
        ],
        out_shape=[
            jax.ShapeDtypeStruct((_NPAD_X, 128), jnp.float32),
            jax.ShapeDtypeStruct((_NPAD_X, 128), jnp.int32),
            jax.ShapeDtypeStruct((_NPAD_X, 128), jnp.float32),
        ],
    )(x_pad, w_all, b_all)

    ep = pl.pallas_call(
        _ep_body,
        grid=(40,),
        in_specs=[
            pl.BlockSpec((8000, 16), lambda i: (i, 0)),
            pl.BlockSpec((16, 128), lambda i: (0, 0)),
        ],
        out_specs=pl.BlockSpec((8000, 64), lambda i: (i, 0)),
        out_shape=jax.ShapeDtypeStruct((_E, 64), jnp.int32),
    )(edge_attr, We)

    ep_pad = jnp.pad(ep, ((0, _EP_PAD - _E), (0, 0)))
    src_pad = jnp.pad(edge_index[0], (0, _IDX_PAD - _E))
    dst_pad = jnp.pad(edge_index[1], (0, _IDX_PAD - _E),
                      constant_values=_TRASH)

    partial, s_out = _edge_kernel(q, kv, ep_pad, src_pad, dst_pad)
    partial = partial[:, :_N]
    s_out = s_out.reshape(_NC, 160 * 128)[:, :2 * _N].reshape(_NC, _N, 2)

    out = pl.pallas_call(
        _fin_body,
        grid=(10,),
        in_specs=[
            pl.BlockSpec((2, 1000, 128), lambda i: (0, i, 0)),
            pl.BlockSpec((2, 1000, 2), lambda i: (0, i, 0)),
            pl.BlockSpec((1000, 128), lambda i: (i, 0)),
        ],
        out_specs=pl.BlockSpec((1000, 128), lambda i: (i, 0)),
        out_shape=jax.ShapeDtypeStruct((_N, _HC), jnp.float32),
    )(partial, s_out, sk[:_N])
    return out


# speedup vs baseline: 51.8951x; 3.2767x over previous
import functools
import math

import jax
import jax.numpy as jnp
from jax import lax
from jax.experimental import pallas as pl
from jax.experimental.pallas import tpu as pltpu
from jax.experimental.pallas import tpu_sc as plsc

_N = 10000
_E = 320000
_DIN = 128
_HC = 128
_NC = 2
_NS = 16
_CH = 16
_NW = _NC * _NS
_EP_PAD = ((_E + _CH * _NW - 1) // (_CH * _NW)) * (_CH * _NW)
_EPW = _EP_PAD // _NW
_IB = 32
_IBE = _IB * _CH
_IDX_PAD = _EP_PAD + _IBE
_NP = ((_N + _CH - 1) // _CH) * _CH
_TRASH = _N + 8
_NPAD_X = 10240


def _pack2(lo_f32, hi_f32):
    lo = lax.bitcast_convert_type(lo_f32.astype(jnp.bfloat16),
                                  jnp.uint16).astype(jnp.int32)
    hi = lax.bitcast_convert_type(hi_f32.astype(jnp.bfloat16),
                                  jnp.uint16).astype(jnp.int32)
    return lax.bitwise_or(lo, lax.shift_left(hi, 16))


def _prep_body(x_ref, w_ref, b_ref, q_ref, kv_ref, sk_ref):
    acc = jnp.dot(x_ref[...], w_ref[...],
                  preferred_element_type=jnp.float32) + b_ref[...]
    q_ref[...] = acc[:, 0:128]
    k = acc[:, 128:256]
    v = acc[:, 256:384]
    kv_ref[...] = jnp.concatenate(
        [_pack2(k[:, 0:64], k[:, 64:128]),
         _pack2(v[:, 0:64], v[:, 64:128])], axis=1)
    sk_ref[...] = acc[:, 384:512]


def _ep_body(a_ref, w_ref, o_ref):
    e = jnp.dot(a_ref[...], w_ref[...], preferred_element_type=jnp.float32)
    o_ref[...] = _pack2(e[:, 0:64], e[:, 64:128])


def _edge_kernel_body(q_hbm, kv_hbm, ep_hbm, src_hbm, dst_hbm,
                      out_hbm, sout_hbm,
                      srcva, dstva, qva, kvva, eva,
                      srcvb, dstvb, qvb, kvvb, evb,
                      srcblk, dstblk,
                      rowv, slocal, idx160, acc, s_acc,
                      sqa, ska, sea, sqb, skb, seb):
    c = lax.axis_index("c")
    s = lax.axis_index("s")
    wid = c * _NS + s
    iot = lax.iota(jnp.int32, 16)

    def _zrow(i, _):
        for k in range(8):
            rowv[i, 16 * k:16 * k + 16] = jnp.zeros((16,), jnp.float32)
        return 0

    lax.fori_loop(0, _CH, _zrow, 0)

    def _zs(i, _):
        for k in range(8):
            slocal[i, 16 * k:16 * k + 16] = jnp.zeros((16,), jnp.float32)
        return 0

    lax.fori_loop(0, 160, _zs, 0)

    def _zi(t, _):
        idx160[pl.ds(t * 16, 16)] = iot + t * 16
        return 0

    lax.fori_loop(0, 10, _zi, 0)

    def _zacc(t, _):
        j = s + t * _NS

        @pl.when(j < _NP // _CH)
        def _():
            rb = pl.multiple_of(j * _CH, 8)
            pltpu.sync_copy(rowv, acc.at[pl.ds(rb, _CH)])

        return 0

    lax.fori_loop(0, _NP // _CH // _NS + 1, _zacc, 0)

    @pl.when(s < 10)
    def _():
        rb = pl.multiple_of(s * 16, 8)
        pltpu.sync_copy(rowv.at[pl.ds(0, 16)], s_acc.at[pl.ds(rb, 16)])

    plsc.subcore_barrier()

    inv_sqrt_c = jnp.float32(1.0 / math.sqrt(64.0))
    bufs_a = (srcva, dstva, qva, kvva, eva, sqa, ska, sea)
    bufs_b = (srcvb, dstvb, qvb, kvvb, evb, sqb, skb, seb)

    def prefetch(tch, bufs):
        srcx, dstx, qx, kvx, ex, sq, sk_, se = bufs
        t_loc = lax.bitwise_and(tch, _IB - 1)

        @pl.when(t_loc == 0)
        def _():
            bb = pl.multiple_of(wid * _EPW + (tch // _IB) * _IBE, 8)
            pltpu.sync_copy(src_hbm.at[pl.ds(bb, _IBE)], srcblk)
            pltpu.sync_copy(dst_hbm.at[pl.ds(bb, _IBE)], dstblk)

        off = pl.multiple_of(t_loc * _CH, 8)
        srcx[pl.ds(0, 16)] = srcblk[pl.ds(off, 16)]
        dstx[pl.ds(0, 16)] = dstblk[pl.ds(off, 16)]
        base = pl.multiple_of(wid * _EPW + tch * _CH, 8)
        pltpu.async_copy(q_hbm.at[dstx], qx, sq)
        pltpu.async_copy(kv_hbm.at[srcx], kvx, sk_)
        pltpu.async_copy(ep_hbm.at[pl.ds(base, _CH)], ex, se)

    def compute(bufs):
        srcx, dstx, qx, kvx, ex, sq, sk_, se = bufs
        pltpu.make_async_copy(q_hbm.at[dstx], qx, sq).wait()
        pltpu.make_async_copy(kv_hbm.at[srcx], kvx, sk_).wait()
        pltpu.make_async_copy(ep_hbm.at[pl.ds(0, _CH)], ex, se).wait()
        for g in range(_CH // 16):
            gb = g * 16
            z16 = jnp.zeros((16,), jnp.float32)

            def _edge_body(i, carry):
                wv0, wv1 = carry
                ii = gb + i
                es = []
                p0 = z16
                p1 = z16
                for j in range(4):
                    kp = kvx[ii, 16 * j:16 * j + 16]
                    epj = ex[ii, 16 * j:16 * j + 16]
                    k0, k1 = plsc.unpack(plsc.bitcast(kp, jnp.bfloat16),
                                         format=plsc.PackFormat.INTERLEAVED)
                    e0, e1 = plsc.unpack(plsc.bitcast(epj, jnp.bfloat16),
                                         format=plsc.PackFormat.INTERLEAVED)
                    q0 = qx[ii, 16 * j:16 * j + 16]
                    q1 = qx[ii, 64 + 16 * j:64 + 16 * j + 16]
                    p0 = p0 + q0 * (k0 + e0)
                    p1 = p1 + q1 * (k1 + e1)
                    es.append((e0, e1))
                a0 = jnp.sum(p0) * inv_sqrt_c
                a1 = jnp.sum(p1) * inv_sqrt_c
                w0 = jnp.exp(jnp.broadcast_to(a0, (16,)))
                w1 = jnp.exp(jnp.broadcast_to(a1, (16,)))
                wv0 = jnp.where(iot == i, w0, wv0)
                wv1 = jnp.where(iot == i, w1, wv1)
                for j in range(4):
                    vp = kvx[ii, 64 + 16 * j:64 + 16 * j + 16]
                    v0, v1 = plsc.unpack(plsc.bitcast(vp, jnp.bfloat16),
                                         format=plsc.PackFormat.INTERLEAVED)
                    e0, e1 = es[j]
                    rowv[ii, 16 * j:16 * j + 16] = (v0 + e0) * w0
                    rowv[ii, 64 + 16 * j:64 + 16 * j + 16] = (v1 + e1) * w1
                return (wv0, wv1)

            wv0, wv1 = plsc.parallel_loop(
                0, 16, unroll=2, carry=(z16, z16))(_edge_body)

            d16 = dstx[pl.ds(gb, 16)]
            f0 = d16 * 2
            row0 = lax.shift_right_logical(f0, 7)
            col0 = lax.bitwise_and(f0, 127)
            plsc.addupdate_scatter(slocal, [row0, col0], wv0)
            plsc.addupdate_scatter(slocal, [row0, col0 + 1], wv1)
        pltpu.sync_copy(rowv, acc.at[dstx], add=True)

    n_per_w = _EP_PAD // _CH // _NW
    prefetch(0, bufs_a)

    def step(t, _):
        prefetch(2 * t + 1, bufs_b)
        compute(bufs_a)
        prefetch(2 * t + 2, bufs_a)
        compute(bufs_b)
        return 0

    lax.fori_loop(0, (n_per_w - 1) // 2, step, 0)
    compute(bufs_a)

    pltpu.sync_copy(slocal, s_acc.at[idx160], add=True)
    plsc.subcore_barrier()

    @pl.when(s < 10)
    def _():
        rb = pl.multiple_of(s * 16, 8)
        pltpu.sync_copy(s_acc.at[pl.ds(rb, 16)],
                        sout_hbm.at[c, pl.ds(rb, 16)])

    def _dump(t, _):
        j = s + t * _NS

        @pl.when(j < _NP // _CH)
        def _():
            rb = pl.multiple_of(j * _CH, 8)
            pltpu.sync_copy(acc.at[pl.ds(rb, _CH)],
                            out_hbm.at[c, pl.ds(rb, _CH)])

        return 0

    lax.fori_loop(0, _NP // _CH // _NS + 1, _dump, 0)


_edge_kernel = functools.partial(
    pl.kernel,
    compiler_params=pltpu.CompilerParams(needs_layout_passes=False),
    out_type=(jax.ShapeDtypeStruct((_NC, _NP, 128), jnp.float32),
              jax.ShapeDtypeStruct((_NC, 160, 128), jnp.float32)),
    mesh=plsc.VectorSubcoreMesh(core_axis_name="c", subcore_axis_name="s"),
    scratch_types=[
        pltpu.VMEM((_CH,), jnp.int32),
        pltpu.VMEM((_CH,), jnp.int32),
        pltpu.VMEM((_CH, 128), jnp.float32),
        pltpu.VMEM((_CH, 128), jnp.int32),
        pltpu.VMEM((_CH, 64), jnp.int32),
        pltpu.VMEM((_CH,), jnp.int32),
        pltpu.VMEM((_CH,), jnp.int32),
        pltpu.VMEM((_CH, 128), jnp.float32),
        pltpu.VMEM((_CH, 128), jnp.int32),
        pltpu.VMEM((_CH, 64), jnp.int32),
        pltpu.VMEM((_IBE,), jnp.int32),
        pltpu.VMEM((_IBE,), jnp.int32),
        pltpu.VMEM((_CH, 128), jnp.float32),
        pltpu.VMEM((160, 128), jnp.float32),
        pltpu.VMEM((160,), jnp.int32),
        pltpu.VMEM_SHARED((_NP, 128), jnp.float32),
        pltpu.VMEM_SHARED((160, 128), jnp.float32),
        pltpu.SemaphoreType.DMA,
        pltpu.SemaphoreType.DMA,
        pltpu.SemaphoreType.DMA,
        pltpu.SemaphoreType.DMA,
        pltpu.SemaphoreType.DMA,
        pltpu.SemaphoreType.DMA,
    ],
)(_edge_kernel_body)


def _fin_body(p_ref, s_ref, sk_ref, o_ref):
    a = p_ref[0] + p_ref[1]
    sv = s_ref[0] + s_ref[1]
    s0 = sv[:, 0:1]
    s1 = sv[:, 1:2]
    o_ref[...] = jnp.concatenate(
        [a[:, 0:64] / (s0 + 1e-16), a[:, 64:128] / (s1 + 1e-16)],
        axis=1) + sk_ref[...]


def kernel(x, edge_index, edge_attr, Wq, bq, Wk, bk, Wv, bv, We, Wskip, bskip):
    w_all = jnp.concatenate([Wq, Wk, Wv, Wskip], axis=1)
    b_all = jnp.concatenate([bq, bk, bv, bskip]).reshape(1, 512)
    x_pad = jnp.pad(x, ((0, _NPAD_X - _N), (0, 0)))

    q, kv, sk = pl.pallas_call(
        _prep_body,
        grid=(5,),
        in_specs=[
            pl.BlockSpec((2048, 128), lambda i: (i, 0)),
            pl.BlockSpec((128, 512), lambda i: (0, 0)),
            pl.BlockSpec((1, 512), lambda i: (0, 0)),
        ],
        out_specs=[
            pl.BlockSpec((2048, 128), lambda i: (i, 0)),
            pl.BlockSpec((2048, 128), lambda i: (i, 0)),
            pl.BlockSpec((2048, 128), lambda i: (i, 0)),
        ],
        out_shape=[
            jax.ShapeDtypeStruct((_NPAD_X, 128), jnp.float32),
            jax.ShapeDtypeStruct((_NPAD_X, 128), jnp.int32),
            jax.ShapeDtypeStruct((_NPAD_X, 128), jnp.float32),
        ],
    )(x_pad, w_all, b_all)

    ep = pl.pallas_call(
        _ep_body,
        grid=(40,),
        in_specs=[
            pl.BlockSpec((8000, 16), lambda i: (i, 0)),
            pl.BlockSpec((16, 128), lambda i: (0, 0)),
        ],
        out_specs=pl.BlockSpec((8000, 64), lambda i: (i, 0)),
        out_shape=jax.ShapeDtypeStruct((_E, 64), jnp.int32),
    )(edge_attr, We)

    ep_pad = jnp.pad(ep, ((0, _EP_PAD - _E), (0, 0)))
    src_pad = jnp.pad(edge_index[0], (0, _IDX_PAD - _E))
    dst_pad = jnp.pad(edge_index[1], (0, _IDX_PAD - _E),
                      constant_values=_TRASH)

    partial, s_out = _edge_kernel(q, kv, ep_pad, src_pad, dst_pad)
    partial = partial[:, :_N]
    s_out = s_out.reshape(_NC, 160 * 128)[:, :2 * _N].reshape(_NC, _N, 2)

    out = pl.pallas_call(
        _fin_body,
        grid=(10,),
        in_specs=[
            pl.BlockSpec((2, 1000, 128), lambda i: (0, i, 0)),
            pl.BlockSpec((2, 1000, 2), lambda i: (0, i, 0)),
            pl.BlockSpec((1000, 128), lambda i: (i, 0)),
        ],
        out_specs=pl.BlockSpec((1000, 128), lambda i: (i, 0)),
        out_shape=jax.ShapeDtypeStruct((_N, _HC), jnp.float32),
    )(partial, s_out, sk[:_N])
    return out


# speedup vs baseline: 54.8259x; 1.0565x over previous
import functools
import math

import jax
import jax.numpy as jnp
from jax import lax
from jax.experimental import pallas as pl
from jax.experimental.pallas import tpu as pltpu
from jax.experimental.pallas import tpu_sc as plsc

_N = 10000
_E = 320000
_DIN = 128
_HC = 128
_NC = 2
_NS = 16
_CH = 16
_NW = _NC * _NS
_EP_PAD = ((_E + _CH * _NW - 1) // (_CH * _NW)) * (_CH * _NW)
_EPW = _EP_PAD // _NW
_IB = 32
_IBE = _IB * _CH
_IDX_PAD = _EP_PAD + _IBE
_NP = ((_N + _CH - 1) // _CH) * _CH
_TRASH = _N + 8
_NPAD_X = 10240


def _pack2(lo_f32, hi_f32):
    lo = lax.bitcast_convert_type(lo_f32.astype(jnp.bfloat16),
                                  jnp.uint16).astype(jnp.int32)
    hi = lax.bitcast_convert_type(hi_f32.astype(jnp.bfloat16),
                                  jnp.uint16).astype(jnp.int32)
    return lax.bitwise_or(lo, lax.shift_left(hi, 16))


def _prep_body(x_ref, w_ref, b_ref, q_ref, kv_ref, sk_ref):
    acc = jnp.dot(x_ref[...], w_ref[...],
                  preferred_element_type=jnp.float32) + b_ref[...]
    q_ref[...] = acc[:, 0:128]
    k = acc[:, 128:256]
    v = acc[:, 256:384]
    kv_ref[...] = jnp.concatenate(
        [_pack2(k[:, 0:64], k[:, 64:128]),
         _pack2(v[:, 0:64], v[:, 64:128])], axis=1)
    sk_ref[...] = acc[:, 384:512]


def _ep_body(a_ref, w_ref, o_ref):
    e = jnp.dot(a_ref[...], w_ref[...], preferred_element_type=jnp.float32)
    o_ref[...] = _pack2(e[:, 0:64], e[:, 64:128])


def _edge_kernel_body(q_hbm, kv_hbm, ep_hbm, src_hbm, dst_hbm,
                      out_hbm, sout_hbm,
                      srcva, dstva, qva, kvva, eva,
                      srcvb, dstvb, qvb, kvvb, evb,
                      srcblk, dstblk,
                      rowv, rowvb, slocal, idx160, acc, s_acc,
                      sqa, ska, sea, sqb, skb, seb, sra, srb):
    c = lax.axis_index("c")
    s = lax.axis_index("s")
    wid = c * _NS + s
    iot = lax.iota(jnp.int32, 16)

    def _zrow(i, _):
        for k in range(8):
            rowv[i, 16 * k:16 * k + 16] = jnp.zeros((16,), jnp.float32)
            rowvb[i, 16 * k:16 * k + 16] = jnp.zeros((16,), jnp.float32)
        return 0

    lax.fori_loop(0, _CH, _zrow, 0)
    srcva[pl.ds(0, 16)] = iot
    dstva[pl.ds(0, 16)] = iot
    srcvb[pl.ds(0, 16)] = iot
    dstvb[pl.ds(0, 16)] = iot

    def _zs(i, _):
        for k in range(8):
            slocal[i, 16 * k:16 * k + 16] = jnp.zeros((16,), jnp.float32)
        return 0

    lax.fori_loop(0, 160, _zs, 0)

    def _zi(t, _):
        idx160[pl.ds(t * 16, 16)] = iot + t * 16
        return 0

    lax.fori_loop(0, 10, _zi, 0)

    def _zacc(t, _):
        j = s + t * _NS

        @pl.when(j < _NP // _CH)
        def _():
            rb = pl.multiple_of(j * _CH, 8)
            pltpu.sync_copy(rowv, acc.at[pl.ds(rb, _CH)])

        return 0

    lax.fori_loop(0, _NP // _CH // _NS + 1, _zacc, 0)

    @pl.when(s < 10)
    def _():
        rb = pl.multiple_of(s * 16, 8)
        pltpu.sync_copy(rowv.at[pl.ds(0, 16)], s_acc.at[pl.ds(rb, 16)])

    plsc.subcore_barrier()

    inv_sqrt_c = jnp.float32(1.0 / math.sqrt(64.0))
    bufs_a = (srcva, dstva, qva, kvva, eva, sqa, ska, sea, rowv, sra)
    bufs_b = (srcvb, dstvb, qvb, kvvb, evb, sqb, skb, seb, rowvb, srb)

    def prefetch(tch, bufs):
        srcx, dstx, qx, kvx, ex, sq, sk_, se, _rowx, _sr = bufs
        t_loc = lax.bitwise_and(tch, _IB - 1)

        @pl.when(t_loc == 0)
        def _():
            bb = pl.multiple_of(wid * _EPW + (tch // _IB) * _IBE, 8)
            pltpu.sync_copy(src_hbm.at[pl.ds(bb, _IBE)], srcblk)
            pltpu.sync_copy(dst_hbm.at[pl.ds(bb, _IBE)], dstblk)

        off = pl.multiple_of(t_loc * _CH, 8)
        srcx[pl.ds(0, 16)] = srcblk[pl.ds(off, 16)]
        dstx[pl.ds(0, 16)] = dstblk[pl.ds(off, 16)]
        base = pl.multiple_of(wid * _EPW + tch * _CH, 8)
        pltpu.async_copy(q_hbm.at[dstx], qx, sq)
        pltpu.async_copy(kv_hbm.at[srcx], kvx, sk_)
        pltpu.async_copy(ep_hbm.at[pl.ds(base, _CH)], ex, se)

    def compute(bufs):
        srcx, dstx, qx, kvx, ex, sq, sk_, se, rowx, sr = bufs
        pltpu.make_async_copy(rowx, acc.at[dstx], sr).wait()
        pltpu.make_async_copy(q_hbm.at[dstx], qx, sq).wait()
        pltpu.make_async_copy(kv_hbm.at[srcx], kvx, sk_).wait()
        pltpu.make_async_copy(ep_hbm.at[pl.ds(0, _CH)], ex, se).wait()
        for g in range(_CH // 16):
            gb = g * 16
            z16 = jnp.zeros((16,), jnp.float32)

            def _edge_body(i, carry):
                wv0, wv1 = carry
                ii = gb + i
                es = []
                p0 = z16
                p1 = z16
                for j in range(4):
                    kp = kvx[ii, 16 * j:16 * j + 16]
                    epj = ex[ii, 16 * j:16 * j + 16]
                    k0, k1 = plsc.unpack(plsc.bitcast(kp, jnp.bfloat16),
                                         format=plsc.PackFormat.INTERLEAVED)
                    e0, e1 = plsc.unpack(plsc.bitcast(epj, jnp.bfloat16),
                                         format=plsc.PackFormat.INTERLEAVED)
                    q0 = qx[ii, 16 * j:16 * j + 16]
                    q1 = qx[ii, 64 + 16 * j:64 + 16 * j + 16]
                    p0 = p0 + q0 * (k0 + e0)
                    p1 = p1 + q1 * (k1 + e1)
                    es.append((e0, e1))
                a0 = jnp.sum(p0) * inv_sqrt_c
                a1 = jnp.sum(p1) * inv_sqrt_c
                w0 = jnp.exp(jnp.broadcast_to(a0, (16,)))
                w1 = jnp.exp(jnp.broadcast_to(a1, (16,)))
                wv0 = jnp.where(iot == i, w0, wv0)
                wv1 = jnp.where(iot == i, w1, wv1)
                for j in range(4):
                    vp = kvx[ii, 64 + 16 * j:64 + 16 * j + 16]
                    v0, v1 = plsc.unpack(plsc.bitcast(vp, jnp.bfloat16),
                                         format=plsc.PackFormat.INTERLEAVED)
                    e0, e1 = es[j]
                    rowx[ii, 16 * j:16 * j + 16] = (v0 + e0) * w0
                    rowx[ii, 64 + 16 * j:64 + 16 * j + 16] = (v1 + e1) * w1
                return (wv0, wv1)

            wv0, wv1 = plsc.parallel_loop(
                0, 16, unroll=2, carry=(z16, z16))(_edge_body)

            d16 = dstx[pl.ds(gb, 16)]
            f0 = d16 * 2
            row0 = lax.shift_right_logical(f0, 7)
            col0 = lax.bitwise_and(f0, 127)
            plsc.addupdate_scatter(slocal, [row0, col0], wv0)
            plsc.addupdate_scatter(slocal, [row0, col0 + 1], wv1)
        pltpu.async_copy(rowx, acc.at[dstx], sr, add=True)

    n_per_w = _EP_PAD // _CH // _NW
    pltpu.async_copy(rowv, acc.at[dstva], sra, add=True)
    pltpu.async_copy(rowvb, acc.at[dstvb], srb, add=True)
    prefetch(0, bufs_a)

    def step(t, _):
        prefetch(2 * t + 1, bufs_b)
        compute(bufs_a)
        prefetch(2 * t + 2, bufs_a)
        compute(bufs_b)
        return 0

    lax.fori_loop(0, (n_per_w - 1) // 2, step, 0)
    compute(bufs_a)
    pltpu.make_async_copy(rowv, acc.at[dstva], sra).wait()
    pltpu.make_async_copy(rowvb, acc.at[dstvb], srb).wait()

    pltpu.sync_copy(slocal, s_acc.at[idx160], add=True)
    plsc.subcore_barrier()

    @pl.when(s < 10)
    def _():
        rb = pl.multiple_of(s * 16, 8)
        pltpu.sync_copy(s_acc.at[pl.ds(rb, 16)],
                        sout_hbm.at[c, pl.ds(rb, 16)])

    def _dump(t, _):
        j = s + t * _NS

        @pl.when(j < _NP // _CH)
        def _():
            rb = pl.multiple_of(j * _CH, 8)
            pltpu.sync_copy(acc.at[pl.ds(rb, _CH)],
                            out_hbm.at[c, pl.ds(rb, _CH)])

        return 0

    lax.fori_loop(0, _NP // _CH // _NS + 1, _dump, 0)


_edge_kernel = functools.partial(
    pl.kernel,
    compiler_params=pltpu.CompilerParams(needs_layout_passes=False),
    out_type=(jax.ShapeDtypeStruct((_NC, _NP, 128), jnp.float32),
              jax.ShapeDtypeStruct((_NC, 160, 128), jnp.float32)),
    mesh=plsc.VectorSubcoreMesh(core_axis_name="c", subcore_axis_name="s"),
    scratch_types=[
        pltpu.VMEM((_CH,), jnp.int32),
        pltpu.VMEM((_CH,), jnp.int32),
        pltpu.VMEM((_CH, 128), jnp.float32),
        pltpu.VMEM((_CH, 128), jnp.int32),
        pltpu.VMEM((_CH, 64), jnp.int32),
        pltpu.VMEM((_CH,), jnp.int32),
        pltpu.VMEM((_CH,), jnp.int32),
        pltpu.VMEM((_CH, 128), jnp.float32),
        pltpu.VMEM((_CH, 128), jnp.int32),
        pltpu.VMEM((_CH, 64), jnp.int32),
        pltpu.VMEM((_IBE,), jnp.int32),
        pltpu.VMEM((_IBE,), jnp.int32),
        pltpu.VMEM((_CH, 128), jnp.float32),
        pltpu.VMEM((_CH, 128), jnp.float32),
        pltpu.VMEM((160, 128), jnp.float32),
        pltpu.VMEM((160,), jnp.int32),
        pltpu.VMEM_SHARED((_NP, 128), jnp.float32),
        pltpu.VMEM_SHARED((160, 128), jnp.float32),
        pltpu.SemaphoreType.DMA,
        pltpu.SemaphoreType.DMA,
        pltpu.SemaphoreType.DMA,
        pltpu.SemaphoreType.DMA,
        pltpu.SemaphoreType.DMA,
        pltpu.SemaphoreType.DMA,
        pltpu.SemaphoreType.DMA,
        pltpu.SemaphoreType.DMA,
    ],
)(_edge_kernel_body)


def _fin_body(p_ref, s_ref, sk_ref, o_ref):
    a = p_ref[0] + p_ref[1]
    sv = s_ref[0] + s_ref[1]
    s0 = sv[:, 0:1]
    s1 = sv[:, 1:2]
    o_ref[...] = jnp.concatenate(
        [a[:, 0:64] / (s0 + 1e-16), a[:, 64:128] / (s1 + 1e-16)],
        axis=1) + sk_ref[...]


def kernel(x, edge_index, edge_attr, Wq, bq, Wk, bk, Wv, bv, We, Wskip, bskip):
    w_all = jnp.concatenate([Wq, Wk, Wv, Wskip], axis=1)
    b_all = jnp.concatenate([bq, bk, bv, bskip]).reshape(1, 512)
    x_pad = jnp.pad(x, ((0, _NPAD_X - _N), (0, 0)))

    q, kv, sk = pl.pallas_call(
        _prep_body,
        grid=(5,),
        in_specs=[
            pl.BlockSpec((2048, 128), lambda i: (i, 0)),
            pl.BlockSpec((128, 512), lambda i: (0, 0)),
            pl.BlockSpec((1, 512), lambda i: (0, 0)),
        ],
        out_specs=[
            pl.BlockSpec((2048, 128), lambda i: (i, 0)),
            pl.BlockSpec((2048, 128), lambda i: (i, 0)),
            pl.BlockSpec((2048, 128), lambda i: (i, 0)),
        ],
        out_shape=[
            jax.ShapeDtypeStruct((_NPAD_X, 128), jnp.float32),
            jax.ShapeDtypeStruct((_NPAD_X, 128), jnp.int32),
            jax.ShapeDtypeStruct((_NPAD_X, 128), jnp.float32),
        ],
    )(x_pad, w_all, b_all)

    ep = pl.pallas_call(
        _ep_body,
        grid=(40,),
        in_specs=[
            pl.BlockSpec((8000, 16), lambda i: (i, 0)),
            pl.BlockSpec((16, 128), lambda i: (0, 0)),
        ],
        out_specs=pl.BlockSpec((8000, 64), lambda i: (i, 0)),
        out_shape=jax.ShapeDtypeStruct((_E, 64), jnp.int32),
    )(edge_attr, We)

    ep_pad = jnp.pad(ep, ((0, _EP_PAD - _E), (0, 0)))
    src_pad = jnp.pad(edge_index[0], (0, _IDX_PAD - _E))
    dst_pad = jnp.pad(edge_index[1], (0, _IDX_PAD - _E),
                      constant_values=_TRASH)

    partial, s_out = _edge_kernel(q, kv, ep_pad, src_pad, dst_pad)
    partial = partial[:, :_N]
    s_out = s_out.reshape(_NC, 160 * 128)[:, :2 * _N].reshape(_NC, _N, 2)

    out = pl.pallas_call(
        _fin_body,
        grid=(10,),
        in_specs=[
            pl.BlockSpec((2, 1000, 128), lambda i: (0, i, 0)),
            pl.BlockSpec((2, 1000, 2), lambda i: (0, i, 0)),
            pl.BlockSpec((1000, 128), lambda i: (i, 0)),
        ],
        out_specs=pl.BlockSpec((1000, 128), lambda i: (i, 0)),
        out_shape=jax.ShapeDtypeStruct((_N, _HC), jnp.float32),
    )(partial, s_out, sk[:_N])
    return out
